# baseline (speedup 1.0000x reference)
import functools

import jax
import jax.numpy as jnp
from jax import lax
from jax.experimental import pallas as pl
from jax.experimental.pallas import tpu as pltpu
from jax.experimental.pallas import tpu_sc as plsc

BATCH = 16384
NUM_CORES = 2
NUM_SUBCORES = 16
NUM_WORKERS = NUM_CORES * NUM_SUBCORES
ROWS_PER_WORKER = BATCH // NUM_WORKERS
VIEW_PER_WORKER = ROWS_PER_WORKER // 2
GATHER_CHUNK = 128
NUM_CHUNKS = ROWS_PER_WORKER // GATHER_CHUNK
LANES = 16


def _splat(vec, lane):
    idx = jnp.full((LANES,), lane, dtype=jnp.int32)
    return jnp.take_along_axis(vec, idx, axis=0, mode="promise_in_bounds")


def _body(feat_hbm, aux_hbm, cent_hbm, out_hbm, aux_v, cent_v, feat_v, acc_v,
          gsem, fsem):
    wid = lax.axis_index("s") * NUM_CORES + lax.axis_index("c")

    pltpu.sync_copy(aux_hbm.at[wid], aux_v)
    fcopy = pltpu.async_copy(
        feat_hbm.at[pl.ds(wid * VIEW_PER_WORKER, VIEW_PER_WORKER)],
        feat_v, fsem)
    gathers = []
    for k in range(NUM_CHUNKS):
        gathers.append(
            pltpu.async_copy(
                cent_hbm.at[aux_v.at[k]],
                cent_v.at[pl.ds(k * GATHER_CHUNK, GATHER_CHUNK)],
                gsem))
    fcopy.wait()
    for g in gathers:
        g.wait()

    zeros = jnp.zeros((LANES,), jnp.float32)

    def tblock(t, accs):
        accs = list(accs)
        for u in range(8):
            hv = aux_v[4 + t, pl.ds(u * LANES, LANES)].astype(jnp.float32)
            for k in range(8):
                v = t * 64 + u * 8 + k
                m0 = _splat(hv, 2 * k)
                m1 = _splat(hv, 2 * k + 1)
                for j in range(8):
                    f = feat_v[v, pl.ds(j * LANES, LANES)]
                    if j < 4:
                        r, jj, m = 2 * v, j, m0
                    else:
                        r, jj, m = 2 * v + 1, j - 4, m1
                    cl = cent_v[r, pl.ds(jj * LANES, LANES)]
                    ch = cent_v[r, pl.ds(64 + jj * LANES, LANES)]
                    d = f - (cl + m * (ch - cl))
                    accs[j] = accs[j] + d * d
        return tuple(accs)

    accs = lax.fori_loop(0, 4, tblock, (zeros,) * 8)
    for j in range(8):
        acc_v[pl.ds(j * LANES, LANES)] = accs[j]
    pltpu.sync_copy(acc_v, out_hbm.at[wid, 0])


@jax.jit
def _center_loss(features, labels, centers):
    feats128 = features.reshape(VIEW_PER_WORKER * NUM_WORKERS, 128)
    cent128 = centers.reshape(-1, 128)
    lab = labels.astype(jnp.int32)
    pidx = (lab // 2).reshape(NUM_WORKERS, NUM_CHUNKS, GATHER_CHUNK)
    par = (lab % 2).reshape(NUM_WORKERS, NUM_CHUNKS, GATHER_CHUNK)
    aux = jnp.concatenate([pidx, par], axis=1)
    mesh = plsc.VectorSubcoreMesh(core_axis_name="c", subcore_axis_name="s")
    partials = pl.kernel(
        _body,
        out_type=jax.ShapeDtypeStruct((NUM_WORKERS, 8, 128), jnp.float32),
        mesh=mesh,
        scratch_types=[
            pltpu.VMEM((2 * NUM_CHUNKS, GATHER_CHUNK), jnp.int32),
            pltpu.VMEM((ROWS_PER_WORKER, 128), jnp.float32),
            pltpu.VMEM((VIEW_PER_WORKER, 128), jnp.float32),
            pltpu.VMEM((128,), jnp.float32),
            pltpu.SemaphoreType.DMA,
            pltpu.SemaphoreType.DMA,
        ],
    )(feats128, aux, cent128)
    return jnp.sum(partials[:, 0, :]) / (2.0 * features.shape[0])


def kernel(features, labels, centers):
    return _center_loss(features, labels, centers)


# speedup vs baseline: 2.6746x; 2.6746x over previous
import functools

import jax
import jax.numpy as jnp
from jax import lax
from jax.experimental import pallas as pl
from jax.experimental.pallas import tpu as pltpu
from jax.experimental.pallas import tpu_sc as plsc

BATCH = 16384
EMB_DIM = 64
NUM_CLASSES = 100000
NUM_CORES = 2
NUM_SUBCORES = 16
NUM_WORKERS = NUM_CORES * NUM_SUBCORES
DIMS_PER_WORKER = EMB_DIM // NUM_WORKERS
FCHUNK = 2048
NUM_FCHUNKS = BATCH // FCHUNK
LANES = 16
VECS_PER_FCHUNK = FCHUNK // LANES


def _body(fT_hbm, lab_hbm, cT_hbm, out_hbm, row_v, lab_v, fbuf0, fbuf1, acc_v,
          rsem, lsem, fsem):
    fbufs = (fbuf0, fbuf1)
    wid = lax.axis_index("s") * NUM_CORES + lax.axis_index("c")
    d0 = wid * DIMS_PER_WORKER

    lcopy = pltpu.async_copy(lab_hbm, lab_v, lsem)
    row_copy = pltpu.async_copy(cT_hbm.at[d0], row_v, rsem)
    feat_copies = [
        pltpu.async_copy(fT_hbm.at[d0, pl.ds(0, FCHUNK)], fbuf0, fsem)
    ]
    lcopy.wait()

    acc = jnp.zeros((LANES,), jnp.float32)
    for k in range(DIMS_PER_WORKER):
        d = d0 + k
        row_copy.wait()
        for c in range(NUM_FCHUNKS):
            g = k * NUM_FCHUNKS + c
            if c + 1 < NUM_FCHUNKS:
                feat_copies.append(
                    pltpu.async_copy(
                        fT_hbm.at[d, pl.ds((c + 1) * FCHUNK, FCHUNK)],
                        fbufs[(g + 1) % 2], fsem))
            elif k + 1 < DIMS_PER_WORKER:
                feat_copies.append(
                    pltpu.async_copy(
                        fT_hbm.at[d + 1, pl.ds(0, FCHUNK)],
                        fbufs[(g + 1) % 2], fsem))
            feat_copies[g].wait()
            fb = fbufs[g % 2]
            base = c * FCHUNK

            def chunk(i, acc, fb=fb, base=base):
                idx = lab_v[pl.ds(base + i * LANES, LANES)]
                cv = plsc.load_gather(row_v, [idx])
                fv = fb[pl.ds(i * LANES, LANES)]
                dv = fv - cv
                return acc + dv * dv

            acc = lax.fori_loop(0, VECS_PER_FCHUNK, chunk, acc, unroll=4)
        if k + 1 < DIMS_PER_WORKER:
            row_copy = pltpu.async_copy(cT_hbm.at[d + 1], row_v, rsem)

    acc_v[...] = acc
    pltpu.sync_copy(acc_v, out_hbm.at[wid])


@jax.jit
def _center_loss(features, labels, centers):
    fT = features.T
    cT = centers.T
    lab = labels.astype(jnp.int32)
    mesh = plsc.VectorSubcoreMesh(core_axis_name="c", subcore_axis_name="s")
    partials = pl.kernel(
        _body,
        out_type=jax.ShapeDtypeStruct((NUM_WORKERS, LANES), jnp.float32),
        mesh=mesh,
        scratch_types=[
            pltpu.VMEM((NUM_CLASSES,), jnp.float32),
            pltpu.VMEM((BATCH,), jnp.int32),
            pltpu.VMEM((FCHUNK,), jnp.float32),
            pltpu.VMEM((FCHUNK,), jnp.float32),
            pltpu.VMEM((LANES,), jnp.float32),
            pltpu.SemaphoreType.DMA,
            pltpu.SemaphoreType.DMA,
            pltpu.SemaphoreType.DMA,
        ],
        compiler_params=pltpu.CompilerParams(needs_layout_passes=False),
    )(fT, lab, cT)
    return jnp.sum(partials) / (2.0 * features.shape[0])


def kernel(features, labels, centers):
    return _center_loss(features, labels, centers)
